# Initial kernel scaffold; baseline (speedup 1.0000x reference)
import jax
import jax.numpy as jnp
from jax.experimental import pallas as pl


def kernel(z, W, emb):
    raise NotImplementedError("write your pallas kernel here")


# speedup vs baseline: 2.3299x; 2.3299x over previous
import functools

import jax
import jax.numpy as jnp
from jax.experimental import pallas as pl

_B, _C_IN, _N_T = 4, 384, 196
_K, _D = 1024, 64
_P = _B * _N_T


def _vq_body(zt_ref, wt_ref, emb_ref, embt_ref, out_ref):
    hi = jax.lax.Precision.HIGHEST
    x = jnp.dot(zt_ref[...].astype(jnp.bfloat16),
                wt_ref[...].astype(jnp.bfloat16),
                preferred_element_type=jnp.float32)
    embt = embt_ref[...]
    g = jnp.dot(x, embt, precision=hi, preferred_element_type=jnp.float32)
    x2 = jnp.sum(x * x, axis=1, keepdims=True)
    e2 = jnp.sum(embt * embt, axis=0, keepdims=True)
    d2 = jnp.maximum(x2 - 2.0 * g + e2, 0.0)
    snorm = jnp.sqrt(d2) / (jnp.sqrt(x2) + jnp.sqrt(e2))
    mval = jnp.min(snorm, axis=1, keepdims=True)
    lane = jax.lax.broadcasted_iota(jnp.int32, (_P, _K), 1)
    midx = jnp.min(jnp.where(snorm == mval, lane, _K), axis=1, keepdims=True)
    onehot = (lane == midx).astype(jnp.float32)
    out_ref[...] = jnp.dot(onehot, emb_ref[...], precision=hi,
                           preferred_element_type=jnp.float32)


@functools.partial(jax.jit, static_argnames=())
def kernel(z, W, emb):
    zt = jnp.transpose(z, (0, 2, 1)).reshape(_P, _C_IN)
    zq = pl.pallas_call(
        _vq_body,
        out_shape=jax.ShapeDtypeStruct((_P, _D), jnp.float32),
    )(zt, W.T, emb, emb.T)
    return jnp.transpose(zq.reshape(_B, _N_T, _D), (0, 2, 1))


# speedup vs baseline: 3.4080x; 1.4627x over previous
import functools

import jax
import jax.numpy as jnp
from jax.experimental import pallas as pl

_B, _C_IN, _N_T = 4, 384, 196
_K, _D = 1024, 64
_P = _B * _N_T


def _vq_body(z_ref, w_ref, emb_ref, embt_hi_ref, embt_lo_ref, out_ref):
    hi = jax.lax.Precision.HIGHEST
    wb = w_ref[...].astype(jnp.bfloat16)
    cols = []
    for b in range(_B):
        zb = z_ref[b].astype(jnp.bfloat16)
        cols.append(jnp.dot(wb, zb, preferred_element_type=jnp.float32))
    ze = jnp.concatenate(cols, axis=1)
    emb = emb_ref[...]
    g = jnp.dot(emb, ze, precision=hi, preferred_element_type=jnp.float32)
    x2 = jnp.sum(ze * ze, axis=0, keepdims=True)
    e2 = jnp.sum(emb * emb, axis=1, keepdims=True)
    d2 = jnp.maximum(x2 - 2.0 * g + e2, 0.0)
    snorm = jnp.sqrt(d2) / (jnp.sqrt(x2) + jnp.sqrt(e2))
    mval = jnp.min(snorm, axis=0, keepdims=True)
    row = jax.lax.broadcasted_iota(jnp.int32, (_K, _P), 0)
    midx = jnp.min(jnp.where(snorm == mval, row, _K), axis=0, keepdims=True)
    onehot = (row == midx).astype(jnp.bfloat16)
    zq = (jnp.dot(embt_hi_ref[...], onehot, preferred_element_type=jnp.float32)
          + jnp.dot(embt_lo_ref[...], onehot, preferred_element_type=jnp.float32))
    for b in range(_B):
        out_ref[b] = zq[:, b * _N_T:(b + 1) * _N_T]


@functools.partial(jax.jit, static_argnames=())
def kernel(z, W, emb):
    embt = emb.T
    embt_hi = embt.astype(jnp.bfloat16)
    embt_lo = (embt - embt_hi.astype(jnp.float32)).astype(jnp.bfloat16)
    return pl.pallas_call(
        _vq_body,
        out_shape=jax.ShapeDtypeStruct((_B, _D, _N_T), jnp.float32),
    )(z, W, emb, embt_hi, embt_lo)


# speedup vs baseline: 3.5946x; 1.0548x over previous
import functools

import jax
import jax.numpy as jnp
from jax.experimental import pallas as pl

_B, _C_IN, _N_T = 4, 384, 196
_K, _D = 1024, 64
_P = _B * _N_T


def _vq_body(z_ref, w_ref, emb_ref, embt_hi_ref, embt_lo_ref, out_ref):
    hi = jax.lax.Precision.HIGHEST
    wb = w_ref[...].astype(jnp.bfloat16)
    cols = []
    for b in range(_B):
        zb = z_ref[b].astype(jnp.bfloat16)
        cols.append(jnp.dot(wb, zb, preferred_element_type=jnp.float32))
    ze = jnp.concatenate(cols, axis=1)
    emb = emb_ref[...]
    g = jnp.dot(emb, ze, precision=hi, preferred_element_type=jnp.float32)
    x2 = jnp.sum(ze * ze, axis=0, keepdims=True)
    e2 = jnp.sum(emb * emb, axis=1, keepdims=True)
    d2 = jnp.maximum(x2 - 2.0 * g + e2, 0.0)
    snorm = jnp.sqrt(d2) / (jnp.sqrt(x2) + jnp.sqrt(e2))
    mval = jnp.min(snorm, axis=0, keepdims=True)
    row = jax.lax.broadcasted_iota(jnp.int32, (_K, _P), 0)
    midx = jnp.min(jnp.where(snorm == mval, row, _K), axis=0, keepdims=True)
    onehot = (row == midx).astype(jnp.bfloat16)
    hilo = jnp.concatenate([embt_hi_ref[...], embt_lo_ref[...]], axis=0)
    r = jnp.dot(hilo, onehot, preferred_element_type=jnp.float32)
    zq = r[:_D] + r[_D:]
    for b in range(_B):
        out_ref[b] = zq[:, b * _N_T:(b + 1) * _N_T]


@functools.partial(jax.jit, static_argnames=())
def kernel(z, W, emb):
    embt = emb.T
    embt_hi = embt.astype(jnp.bfloat16)
    embt_lo = (embt - embt_hi.astype(jnp.float32)).astype(jnp.bfloat16)
    return pl.pallas_call(
        _vq_body,
        out_shape=jax.ShapeDtypeStruct((_B, _D, _N_T), jnp.float32),
    )(z, W, emb, embt_hi, embt_lo)
